# baseline (speedup 1.0000x reference)
import functools

import jax
import jax.numpy as jnp
from jax import lax
from jax.experimental import pallas as pl
from jax.experimental.pallas import tpu as pltpu
from jax.experimental.pallas import tpu_sc as plsc

VOCAB = 100001
EMBED_DIM = 32
BATCH = 16384

CHUNK = 128


def _make_gather():
    info = plsc.get_sparse_core_info()
    nw = info.num_cores * info.num_subcores
    b_per_w = BATCH // nw
    n_chunks = b_per_w // CHUNK

    mesh = plsc.VectorSubcoreMesh(core_axis_name="c", subcore_axis_name="s")

    @functools.partial(
        pl.kernel,
        mesh=mesh,
        out_type=jax.ShapeDtypeStruct((BATCH, EMBED_DIM), jnp.float32),
        scratch_types=[
            pltpu.VMEM((b_per_w,), jnp.int32),
            pltpu.VMEM((b_per_w, EMBED_DIM), jnp.float32),
            pltpu.SemaphoreType.DMA,
        ],
        compiler_params=pltpu.CompilerParams(
            use_tc_tiling_on_sc=False,
            skip_device_barrier=True,
            disable_bounds_checks=True,
            disable_semaphore_checks=True,
        ),
    )
    def gather(idx_hbm, table_hbm, out_hbm, idx_v, rows_v, sem):
        wid = lax.axis_index("s") * info.num_cores + lax.axis_index("c")
        base = wid * b_per_w
        pltpu.sync_copy(idx_hbm.at[pl.ds(base, b_per_w)], idx_v)
        copies = [
            pltpu.async_copy(
                table_hbm.at[idx_v.at[pl.ds(c * CHUNK, CHUNK)]],
                rows_v.at[pl.ds(c * CHUNK, CHUNK)],
                sem,
            )
            for c in range(n_chunks)
        ]
        for cp in copies:
            cp.wait()
        pltpu.sync_copy(rows_v, out_hbm.at[pl.ds(base, b_per_w)])

    return gather


_gather = _make_gather()


def kernel(member_id, table):
    return _gather(member_id.astype(jnp.int32), table)


# speedup vs baseline: 1.1076x; 1.1076x over previous
import functools

import jax
import jax.numpy as jnp
from jax import lax
from jax.experimental import pallas as pl
from jax.experimental.pallas import tpu as pltpu
from jax.experimental.pallas import tpu_sc as plsc

VOCAB = 100001
EMBED_DIM = 32
BATCH = 16384

CHUNK = 128
WINDOW = 32


def _make_gather():
    info = plsc.get_sparse_core_info()
    nw = info.num_cores * info.num_subcores
    b_per_w = BATCH // nw
    n_chunks = b_per_w // CHUNK

    mesh = plsc.VectorSubcoreMesh(core_axis_name="c", subcore_axis_name="s")

    @functools.partial(
        pl.kernel,
        mesh=mesh,
        out_type=jax.ShapeDtypeStruct((EMBED_DIM, BATCH), jnp.float32),
        scratch_types=[
            pltpu.VMEM((b_per_w,), jnp.int32),
            pltpu.VMEM((EMBED_DIM, b_per_w), jnp.float32),
            pltpu.SemaphoreType.DMA,
        ],
        compiler_params=pltpu.CompilerParams(use_tc_tiling_on_sc=False),
    )
    def gather(idx_hbm, tab_t_hbm, o_hbm, ridx_v, vt, sem):
        wid = lax.axis_index("s") * info.num_cores + lax.axis_index("c")
        obase = wid * b_per_w
        pltpu.sync_copy(idx_hbm.at[pl.ds(obase, b_per_w)], ridx_v)

        copies = []
        for c in range(EMBED_DIM):
            for h in range(n_chunks):
                if len(copies) >= WINDOW:
                    copies[len(copies) - WINDOW].wait()
                copies.append(pltpu.async_copy(
                    tab_t_hbm.at[c].at[ridx_v.at[pl.ds(h * CHUNK, CHUNK)]],
                    vt.at[c, pl.ds(h * CHUNK, CHUNK)],
                    sem,
                ))
        for cp in copies[len(copies) - WINDOW:]:
            cp.wait()

        pltpu.sync_copy(vt, o_hbm.at[:, pl.ds(obase, b_per_w)])

    return gather


_gather = _make_gather()


def kernel(member_id, table):
    ids = member_id.astype(jnp.int32)
    o = _gather(ids, table.T)
    return o.T


# speedup vs baseline: 1.1918x; 1.0761x over previous
import functools

import jax
import jax.numpy as jnp
from jax import lax
from jax.experimental import pallas as pl
from jax.experimental.pallas import tpu as pltpu
from jax.experimental.pallas import tpu_sc as plsc

VOCAB = 100001
EMBED_DIM = 32
BATCH = 16384

CHUNK = 512
WINDOW = 16


def _make_gather():
    info = plsc.get_sparse_core_info()
    nw = info.num_cores * info.num_subcores
    b_per_w = BATCH // nw
    n_chunks = b_per_w // CHUNK

    mesh = plsc.VectorSubcoreMesh(core_axis_name="c", subcore_axis_name="s")

    @functools.partial(
        pl.kernel,
        mesh=mesh,
        out_type=jax.ShapeDtypeStruct((EMBED_DIM, BATCH), jnp.float32),
        scratch_types=[
            pltpu.VMEM((b_per_w,), jnp.int32),
            pltpu.VMEM((EMBED_DIM, b_per_w), jnp.float32),
            pltpu.SemaphoreType.DMA,
        ],
        compiler_params=pltpu.CompilerParams(use_tc_tiling_on_sc=False),
    )
    def gather(idx_hbm, tab_t_hbm, o_hbm, ridx_v, vt, sem):
        wid = lax.axis_index("s") * info.num_cores + lax.axis_index("c")
        obase = wid * b_per_w
        pltpu.sync_copy(idx_hbm.at[pl.ds(obase, b_per_w)], ridx_v)

        copies = []
        for c in range(EMBED_DIM):
            for h in range(n_chunks):
                if len(copies) >= WINDOW:
                    copies[len(copies) - WINDOW].wait()
                copies.append(pltpu.async_copy(
                    tab_t_hbm.at[c].at[ridx_v.at[pl.ds(h * CHUNK, CHUNK)]],
                    vt.at[c, pl.ds(h * CHUNK, CHUNK)],
                    sem,
                ))
        for cp in copies[len(copies) - WINDOW:]:
            cp.wait()

        pltpu.sync_copy(vt, o_hbm.at[:, pl.ds(obase, b_per_w)])

    return gather


_gather = _make_gather()


def kernel(member_id, table):
    ids = member_id.astype(jnp.int32)
    o = _gather(ids, table.T)
    return o.T


# speedup vs baseline: 1.3316x; 1.1172x over previous
import functools

import jax
import jax.numpy as jnp
from jax import lax
from jax.experimental import pallas as pl
from jax.experimental.pallas import tpu as pltpu
from jax.experimental.pallas import tpu_sc as plsc

VOCAB = 100001
VOCAB_P = 100096
EMBED_DIM = 32
BATCH = 16384

CHUNK = 512
WINDOW = 16


def _make_gather():
    info = plsc.get_sparse_core_info()
    nw = info.num_cores * info.num_subcores
    b_per_w = BATCH // nw
    n_chunks = b_per_w // CHUNK

    mesh = plsc.VectorSubcoreMesh(core_axis_name="c", subcore_axis_name="s")

    @functools.partial(
        pl.kernel,
        mesh=mesh,
        out_type=jax.ShapeDtypeStruct((EMBED_DIM, BATCH), jnp.float32),
        scratch_types=[
            pltpu.VMEM((b_per_w,), jnp.int32),
            pltpu.VMEM((EMBED_DIM, b_per_w), jnp.float32),
            pltpu.SemaphoreType.DMA,
        ],
        compiler_params=pltpu.CompilerParams(use_tc_tiling_on_sc=False),
    )
    def gather(idx_hbm, tab_t_hbm, o_hbm, ridx_v, vt, sem):
        wid = lax.axis_index("s") * info.num_cores + lax.axis_index("c")
        obase = wid * b_per_w
        pltpu.sync_copy(idx_hbm.at[pl.ds(obase, b_per_w)], ridx_v)

        copies = []
        for c in range(EMBED_DIM):
            for h in range(n_chunks):
                if len(copies) >= WINDOW:
                    copies[len(copies) - WINDOW].wait()
                copies.append(pltpu.async_copy(
                    tab_t_hbm.at[c].at[ridx_v.at[pl.ds(h * CHUNK, CHUNK)]],
                    vt.at[c, pl.ds(h * CHUNK, CHUNK)],
                    sem,
                ))
        for cp in copies[len(copies) - WINDOW:]:
            cp.wait()

        pltpu.sync_copy(vt, o_hbm.at[:, pl.ds(obase, b_per_w)])

    return gather


_gather = _make_gather()


def kernel(member_id, table):
    ids = member_id.astype(jnp.int32)
    tab_p = jnp.pad(table.T, ((0, 0), (0, VOCAB_P - VOCAB)))
    o = _gather(ids, tab_p)
    return o.T


# speedup vs baseline: 1.3839x; 1.0393x over previous
import functools

import jax
import jax.numpy as jnp
from jax import lax
from jax.experimental import pallas as pl
from jax.experimental.pallas import tpu as pltpu
from jax.experimental.pallas import tpu_sc as plsc

VOCAB = 100001
VOCAB_P = 100096
EMBED_DIM = 32
BATCH = 16384

CHUNK = 512
WINDOW = 32


def _make_gather():
    info = plsc.get_sparse_core_info()
    nw = info.num_cores * info.num_subcores
    b_per_w = BATCH // nw
    n_chunks = b_per_w // CHUNK

    mesh = plsc.VectorSubcoreMesh(core_axis_name="c", subcore_axis_name="s")

    @functools.partial(
        pl.kernel,
        mesh=mesh,
        out_type=jax.ShapeDtypeStruct((EMBED_DIM, BATCH), jnp.float32),
        scratch_types=[
            pltpu.VMEM((b_per_w,), jnp.int32),
            pltpu.VMEM((EMBED_DIM, b_per_w), jnp.float32),
            pltpu.SemaphoreType.DMA,
        ],
        compiler_params=pltpu.CompilerParams(use_tc_tiling_on_sc=False),
    )
    def gather(idx_hbm, tab_t_hbm, o_hbm, ridx_v, vt, sem):
        wid = lax.axis_index("s") * info.num_cores + lax.axis_index("c")
        obase = wid * b_per_w
        pltpu.sync_copy(idx_hbm.at[pl.ds(obase, b_per_w)], ridx_v)

        copies = []
        for c in range(EMBED_DIM):
            for h in range(n_chunks):
                if len(copies) >= WINDOW:
                    copies[len(copies) - WINDOW].wait()
                copies.append(pltpu.async_copy(
                    tab_t_hbm.at[c].at[ridx_v.at[pl.ds(h * CHUNK, CHUNK)]],
                    vt.at[c, pl.ds(h * CHUNK, CHUNK)],
                    sem,
                ))
        for cp in copies[len(copies) - WINDOW:]:
            cp.wait()

        pltpu.sync_copy(vt, o_hbm.at[:, pl.ds(obase, b_per_w)])

    return gather


_gather = _make_gather()


def kernel(member_id, table):
    ids = member_id.astype(jnp.int32)
    tab_p = jnp.pad(table.T, ((0, 0), (0, VOCAB_P - VOCAB)))
    o = _gather(ids, tab_p)
    return o.T
